# baseline (speedup 1.0000x reference)
import jax
import jax.numpy as jnp
from jax.experimental import pallas as pl
from jax.experimental.pallas import tpu as pltpu

_STAGE_ROWS = 256
_N_STAGES = 4


def kernel(ref_tensor, table):
    batch, _ = ref_tensor.shape
    dim = table.shape[1]
    n_copies = batch // _STAGE_ROWS

    def body(table_ref, out_ref, *stages_and_sem):
        stages, sem = stages_and_sem[:-1], stages_and_sem[-1]
        for st in stages:
            st[:, :] = jnp.broadcast_to(table_ref[:, :], st.shape)
        for i in range(n_copies):
            pltpu.make_async_copy(
                stages[i % _N_STAGES],
                out_ref.at[pl.ds(i * _STAGE_ROWS, _STAGE_ROWS)],
                sem,
            ).start()
        pltpu.make_async_copy(out_ref, out_ref, sem).wait()

    return pl.pallas_call(
        body,
        in_specs=[pl.BlockSpec(memory_space=pltpu.VMEM)],
        out_specs=pl.BlockSpec(memory_space=pltpu.MemorySpace.HBM),
        out_shape=jax.ShapeDtypeStruct((batch, dim), table.dtype),
        scratch_shapes=[
            pltpu.VMEM((_STAGE_ROWS, dim), jnp.float32)
            for _ in range(_N_STAGES)
        ]
        + [pltpu.SemaphoreType.DMA],
    )(table)


# speedup vs baseline: 1.0042x; 1.0042x over previous
import jax
import jax.numpy as jnp
from jax.experimental import pallas as pl
from jax.experimental.pallas import tpu as pltpu

_STAGE_ROWS = 256
_N_STAGES = 4


def kernel(ref_tensor, table):
    batch, _ = ref_tensor.shape
    dim = table.shape[1]
    n_copies = batch // _STAGE_ROWS

    def body(table_ref, out_ref, *stages_and_sem):
        stages, sem = stages_and_sem[:-1], stages_and_sem[-1]
        for st in stages:
            st[:, :] = jnp.broadcast_to(table_ref[:, :], st.shape)
        for i in range(n_copies):
            pltpu.make_async_copy(
                stages[i % _N_STAGES],
                out_ref.at[pl.ds(i * _STAGE_ROWS, _STAGE_ROWS)],
                sem,
            ).start()
        pltpu.make_async_copy(out_ref, out_ref, sem).wait()

    return pl.pallas_call(
        body,
        compiler_params=pltpu.CompilerParams(
            disable_bounds_checks=True,
            disable_semaphore_checks=True,
            skip_device_barrier=True,
        ),
        in_specs=[pl.BlockSpec(memory_space=pltpu.VMEM)],
        out_specs=pl.BlockSpec(memory_space=pltpu.MemorySpace.HBM),
        out_shape=jax.ShapeDtypeStruct((batch, dim), table.dtype),
        scratch_shapes=[
            pltpu.VMEM((_STAGE_ROWS, dim), jnp.float32)
            for _ in range(_N_STAGES)
        ]
        + [pltpu.SemaphoreType.DMA],
    )(table)
